# Initial kernel scaffold; baseline (speedup 1.0000x reference)
import jax
import jax.numpy as jnp
from jax.experimental import pallas as pl


def kernel(x, gate_w, gate_b, A_stack, B_stack, ln_gamma, ln_beta):
    raise NotImplementedError("write your pallas kernel here")


# speedup vs baseline: 1.6214x; 1.6214x over previous
import functools

import jax
import jax.numpy as jnp
from jax.experimental import pallas as pl

B, L, D = 4, 4096, 2048
E, R, K = 8, 8, 2
ALPHA = 1.0 / R
LCHUNK = 512
NL = L // LCHUNK


def _mean_logits_kernel(x_ref, gw_ref, gb_ref, hsum_ref, logits_ref):
    li = pl.program_id(1)

    @pl.when(li == 0)
    def _():
        hsum_ref[...] = jnp.zeros_like(hsum_ref)

    hsum_ref[0] += jnp.sum(x_ref[0], axis=0, keepdims=True)

    @pl.when(li == NL - 1)
    def _():
        h = hsum_ref[0] * (1.0 / L)
        hsum_ref[0] = h
        logits_ref[0] = (
            jnp.dot(h, gw_ref[...].T, preferred_element_type=jnp.float32)
            + gb_ref[...]
        )


def _routing_kernel(h_ref, logits_ref, a_ref, b_ref, delta_ref, aux_ref):
    logits = logits_ref[...]
    col = jax.lax.broadcasted_iota(jnp.int32, (B, E), 1)
    m1 = jnp.max(logits, axis=1, keepdims=True)
    i1 = jnp.min(jnp.where(logits == m1, col, E), axis=1, keepdims=True)
    masked = jnp.where(col == i1, -jnp.inf, logits)
    m2 = jnp.max(masked, axis=1, keepdims=True)
    i2 = jnp.min(jnp.where(masked == m2, col, E), axis=1, keepdims=True)
    w1 = 1.0 / (1.0 + jnp.exp(m2 - m1))
    w2 = 1.0 - w1
    wdense = jnp.where(col == i1, w1, 0.0) + jnp.where(col == i2, w2, 0.0)
    counts = jnp.sum(
        (col == i1).astype(jnp.float32) + (col == i2).astype(jnp.float32),
        axis=0)
    f = counts / (B * K)
    mx = jnp.max(logits, axis=1, keepdims=True)
    ex = jnp.exp(logits - mx)
    sm = ex / jnp.sum(ex, axis=1, keepdims=True)
    P = jnp.mean(sm, axis=0)
    load_balance = E * jnp.sum(f * P)
    log_z = jnp.log(jnp.sum(ex, axis=1)) + mx[:, 0]
    z_loss = jnp.mean(log_z ** 2)
    p = f + 1e-8
    p = p / jnp.sum(p)
    renyi_loss = jnp.log(jnp.sum(p * p))
    aux = 0.01 * load_balance + 0.001 * z_loss + 0.01 * renyi_loss
    aux_ref[...] = jnp.full_like(aux_ref, aux)
    z = jnp.dot(h_ref[...], a_ref[...].T,
                preferred_element_type=jnp.float32)
    coef = jnp.repeat(wdense * ALPHA, R, axis=1)
    delta_ref[...] = jnp.dot(z * coef, b_ref[...],
                             preferred_element_type=jnp.float32)


def _ln_kernel(x_ref, delta_ref, g_ref, bta_ref, o_ref):
    y = x_ref[0] + delta_ref[0]
    mu = jnp.mean(y, axis=1, keepdims=True)
    yc = y - mu
    var = jnp.mean(yc * yc, axis=1, keepdims=True)
    o_ref[0] = yc * jax.lax.rsqrt(var + 1e-5) * g_ref[...] + bta_ref[...]


@jax.jit
def kernel(x, gate_w, gate_b, A_stack, B_stack, ln_gamma, ln_beta):
    h, logits = pl.pallas_call(
        _mean_logits_kernel,
        grid=(B, NL),
        in_specs=[
            pl.BlockSpec((1, LCHUNK, D), lambda b, l: (b, l, 0)),
            pl.BlockSpec((E, D), lambda b, l: (0, 0)),
            pl.BlockSpec((1, E), lambda b, l: (0, 0)),
        ],
        out_specs=[
            pl.BlockSpec((1, 1, D), lambda b, l: (b, 0, 0)),
            pl.BlockSpec((1, 1, E), lambda b, l: (b, 0, 0)),
        ],
        out_shape=[
            jax.ShapeDtypeStruct((B, 1, D), jnp.float32),
            jax.ShapeDtypeStruct((B, 1, E), jnp.float32),
        ],
    )(x, gate_w, gate_b.reshape(1, E))
    h = h.reshape(B, D)
    logits = logits.reshape(B, E)

    a_flat = A_stack.reshape(E * R, D)
    b_flat = B_stack.transpose(0, 2, 1).reshape(E * R, D)
    delta, aux = pl.pallas_call(
        _routing_kernel,
        out_shape=[
            jax.ShapeDtypeStruct((B, D), jnp.float32),
            jax.ShapeDtypeStruct((8, 128), jnp.float32),
        ],
    )(h, logits, a_flat, b_flat)

    out = pl.pallas_call(
        _ln_kernel,
        grid=(B, NL),
        in_specs=[
            pl.BlockSpec((1, LCHUNK, D), lambda b, l: (b, l, 0)),
            pl.BlockSpec((1, 1, D), lambda b, l: (b, 0, 0)),
            pl.BlockSpec((1, D), lambda b, l: (0, 0)),
            pl.BlockSpec((1, D), lambda b, l: (0, 0)),
        ],
        out_specs=pl.BlockSpec((1, LCHUNK, D), lambda b, l: (b, l, 0)),
        out_shape=jax.ShapeDtypeStruct((B, L, D), jnp.float32),
    )(x, delta.reshape(B, 1, D), ln_gamma.reshape(1, D),
      ln_beta.reshape(1, D))

    return out, aux[0, 0]
